# Initial kernel scaffold; baseline (speedup 1.0000x reference)
import jax
import jax.numpy as jnp
from jax.experimental import pallas as pl


def kernel(x, edge_index, Wl0, Wr0, b0, Wl1, Wr1, b1, Wl2, Wr2, b2):
    raise NotImplementedError("write your pallas kernel here")


# speedup vs baseline: 5.1990x; 5.1990x over previous
import functools

import jax
import jax.numpy as jnp
from jax import lax
from jax.experimental import pallas as pl
from jax.experimental.pallas import tpu as pltpu
from jax.experimental.pallas import tpu_sc as plsc

N = 10000
E = 320000
D_IN = 128
D = 256
NP = 10240
ROWS_PER_TILE = NP // 16
CHUNK = 128

_mesh = plsc.VectorSubcoreMesh(core_axis_name="c", subcore_axis_name="s")


def _zero_block(zb):
    z = jnp.zeros((16,), jnp.float32)
    for r in range(16):
        for k in range(8):
            zb[r, pl.ds(k * 16, 16)] = z


def _zero_shared(zb, acc, s):
    def body(k, _):
        pltpu.sync_copy(zb, acc.at[pl.ds(s * ROWS_PER_TILE + k * 16, 16)])
        return 0
    lax.fori_loop(0, ROWS_PER_TILE // 16, body, 0)


def _agg0_body(x_hbm, src_hbm, dst_hbm, agg_hbm, cnt_hbm,
               idx_v, dst_v, rows_v, ones_v, idx_t, dst_t, rows_t,
               zb, zc, acc, cacc, sem):
    c = lax.axis_index("c")
    s = lax.axis_index("s")

    _zero_block(zb)
    _zero_shared(zb, acc, s)
    z = jnp.zeros((16,), jnp.float32)
    o = jnp.ones((16,), jnp.float32)
    for k in range(ROWS_PER_TILE // 16):
        zc[pl.ds(k * 16, 16)] = z
    for k in range(CHUNK // 16):
        ones_v[pl.ds(k * 16, 16)] = o
    pltpu.sync_copy(zc, cacc.at[pl.ds(s * ROWS_PER_TILE, ROWS_PER_TILE)])
    plsc.subcore_barrier()

    base_e = (c * 16 + s) * (E // 32)

    def body(t, _):
        off = base_e + t * CHUNK
        pltpu.sync_copy(src_hbm.at[pl.ds(off, CHUNK)], idx_v)
        pltpu.sync_copy(dst_hbm.at[pl.ds(off, CHUNK)], dst_v)
        pltpu.async_copy(x_hbm.at[idx_v], rows_v, sem).wait()
        pltpu.sync_copy(rows_v, acc.at[dst_v], add=True)
        pltpu.sync_copy(ones_v, cacc.at[dst_v], add=True)
        return 0

    lax.fori_loop(0, (E // 32) // CHUNK, body, 0)

    off = base_e + ((E // 32) // CHUNK) * CHUNK
    pltpu.sync_copy(src_hbm.at[pl.ds(off, 16)], idx_t)
    pltpu.sync_copy(dst_hbm.at[pl.ds(off, 16)], dst_t)
    pltpu.async_copy(x_hbm.at[idx_t], rows_t, sem).wait()
    pltpu.sync_copy(rows_t, acc.at[dst_t], add=True)
    pltpu.sync_copy(ones_v.at[pl.ds(0, 16)], cacc.at[dst_t], add=True)

    plsc.subcore_barrier()
    r0 = s * ROWS_PER_TILE
    pltpu.sync_copy(acc.at[pl.ds(r0, ROWS_PER_TILE)],
                    agg_hbm.at[pl.ds(c * NP + r0, ROWS_PER_TILE)])
    pltpu.sync_copy(cacc.at[pl.ds(r0, ROWS_PER_TILE)],
                    cnt_hbm.at[pl.ds(c * NP + r0, ROWS_PER_TILE)])


_sc_agg0 = pl.kernel(
    _agg0_body,
    out_type=(jax.ShapeDtypeStruct((2 * NP, 128), jnp.float32),
              jax.ShapeDtypeStruct((2 * NP,), jnp.float32)),
    mesh=_mesh,
    scratch_types=[
        pltpu.VMEM((CHUNK,), jnp.int32),
        pltpu.VMEM((CHUNK,), jnp.int32),
        pltpu.VMEM((CHUNK, 128), jnp.float32),
        pltpu.VMEM((CHUNK,), jnp.float32),
        pltpu.VMEM((16,), jnp.int32),
        pltpu.VMEM((16,), jnp.int32),
        pltpu.VMEM((16, 128), jnp.float32),
        pltpu.VMEM((16, 128), jnp.float32),
        pltpu.VMEM((ROWS_PER_TILE,), jnp.float32),
        pltpu.VMEM_SHARED((NP, 128), jnp.float32),
        pltpu.VMEM_SHARED((NP,), jnp.float32),
        pltpu.SemaphoreType.DMA,
    ],
)


def _agg_body(h_hbm, src_hbm, dst_hbm, agg_hbm,
              idx_v, dst_v, rows_v, idx_t, dst_t, rows_t,
              zb, acc, sem):
    c = lax.axis_index("c")
    s = lax.axis_index("s")

    _zero_block(zb)
    _zero_shared(zb, acc, s)
    plsc.subcore_barrier()

    base_e = s * (E // 16)

    def body(t, _):
        off = c * E + base_e + t * CHUNK
        doff = base_e + t * CHUNK
        pltpu.sync_copy(src_hbm.at[pl.ds(off, CHUNK)], idx_v)
        pltpu.sync_copy(dst_hbm.at[pl.ds(doff, CHUNK)], dst_v)
        pltpu.async_copy(h_hbm.at[idx_v], rows_v, sem).wait()
        pltpu.sync_copy(rows_v, acc.at[dst_v], add=True)
        return 0

    lax.fori_loop(0, (E // 16) // CHUNK, body, 0)

    toff = base_e + ((E // 16) // CHUNK) * CHUNK
    pltpu.sync_copy(src_hbm.at[pl.ds(c * E + toff, 32)], idx_t)
    pltpu.sync_copy(dst_hbm.at[pl.ds(toff, 32)], dst_t)
    pltpu.async_copy(h_hbm.at[idx_t], rows_t, sem).wait()
    pltpu.sync_copy(rows_t, acc.at[dst_t], add=True)

    plsc.subcore_barrier()
    r0 = s * ROWS_PER_TILE
    pltpu.sync_copy(acc.at[pl.ds(r0, ROWS_PER_TILE)],
                    agg_hbm.at[pl.ds(c * NP + r0, ROWS_PER_TILE)])


_sc_agg = pl.kernel(
    _agg_body,
    out_type=jax.ShapeDtypeStruct((2 * NP, 128), jnp.float32),
    mesh=_mesh,
    scratch_types=[
        pltpu.VMEM((CHUNK,), jnp.int32),
        pltpu.VMEM((CHUNK,), jnp.int32),
        pltpu.VMEM((CHUNK, 128), jnp.float32),
        pltpu.VMEM((32,), jnp.int32),
        pltpu.VMEM((32,), jnp.int32),
        pltpu.VMEM((32, 128), jnp.float32),
        pltpu.VMEM((16, 128), jnp.float32),
        pltpu.VMEM_SHARED((NP, 128), jnp.float32),
        pltpu.SemaphoreType.DMA,
    ],
)


ROW_BLK = 2048


def _tc_layer_body(nparts_in, parts_out,
                   agg_ref, cnt_ref, h_ref, wl_ref, wr_ref, b_ref, o_ref):
    cnt = cnt_ref[0] + cnt_ref[1]
    inv = 1.0 / jnp.maximum(cnt, 1.0)
    acc = jnp.zeros((ROW_BLK, 128), jnp.float32)
    for c in range(2):
        acc = acc + jnp.dot(agg_ref[c] * inv[:, None], wl_ref[c],
                            preferred_element_type=jnp.float32)
    for q in range(nparts_in):
        acc = acc + jnp.dot(h_ref[q], wr_ref[q],
                            preferred_element_type=jnp.float32)
    acc = acc + b_ref[0][None, :]
    out = jnp.maximum(acc, 0.0)
    if parts_out:
        o_ref[...] = out[None]
    else:
        o_ref[...] = out


def _tc_layer(agg, cnt, h_parts, wlt, wrt, b, parts_out):
    p_in = h_parts.shape[0]
    grid = (5, 2)
    if parts_out:
        out_shape = jax.ShapeDtypeStruct((2, NP, 128), jnp.float32)
        out_spec = pl.BlockSpec((1, ROW_BLK, 128), lambda i, p: (p, i, 0))
    else:
        out_shape = jax.ShapeDtypeStruct((N, D), jnp.float32)
        out_spec = pl.BlockSpec((ROW_BLK, 128), lambda i, p: (i, p))
    return pl.pallas_call(
        functools.partial(_tc_layer_body, p_in, parts_out),
        grid=grid,
        in_specs=[
            pl.BlockSpec((2, ROW_BLK, 128), lambda i, p: (0, i, 0)),
            pl.BlockSpec((2, ROW_BLK), lambda i, p: (0, i)),
            pl.BlockSpec((p_in, ROW_BLK, 128), lambda i, p: (0, i, 0)),
            pl.BlockSpec((2, 128, 128), lambda i, p: (0, 0, p)),
            pl.BlockSpec((p_in, 128, 128), lambda i, p: (0, 0, p)),
            pl.BlockSpec((1, 128), lambda i, p: (0, p)),
        ],
        out_specs=out_spec,
        out_shape=out_shape,
    )(agg, cnt, h_parts, wlt, wrt, b)


def kernel(x, edge_index, Wl0, Wr0, b0, Wl1, Wr1, b1, Wl2, Wr2, b2):
    src = edge_index[0]
    dst = edge_index[1]
    src_both = jnp.concatenate([src, src + NP])

    agg0, cnt = _sc_agg0(x, src, dst)
    agg0 = agg0.reshape(2, NP, 128)
    cnt = cnt.reshape(2, NP)
    h1 = _tc_layer(agg0, cnt, x.reshape(1, N, 128),
                   jnp.stack([Wl0.T, Wl0.T]), Wr0.T.reshape(1, 128, D),
                   b0.reshape(1, D), parts_out=True)

    agg1 = _sc_agg(h1.reshape(2 * NP, 128), src_both, dst).reshape(2, NP, 128)
    h2 = _tc_layer(agg1, cnt, h1,
                   Wl1.T.reshape(2, 128, D), Wr1.T.reshape(2, 128, D),
                   b1.reshape(1, D), parts_out=True)

    agg2 = _sc_agg(h2.reshape(2 * NP, 128), src_both, dst).reshape(2, NP, 128)
    h3 = _tc_layer(agg2, cnt, h2,
                   Wl2.T.reshape(2, 128, D), Wr2.T.reshape(2, 128, D),
                   b2.reshape(1, D), parts_out=False)

    return h3.reshape(1, N, D)
